# baseline (speedup 1.0000x reference)
import functools

import jax
import jax.numpy as jnp
from jax import lax
from jax.experimental import pallas as pl
from jax.experimental.pallas import tpu as pltpu
from jax.experimental.pallas import tpu_sc as plsc

N_NODES = 10000
N_EDGES = 160000
IN_FEATS = 16
HID = 16
NODE_FEAT = 32

NC = 2
NS = 16
NW = NC * NS
CH = 128
KCH = 40
E_PAD = NW * KCH * CH
EPW = KCH * CH
N_PAD = 10240
ROWS_PER_SUB = N_PAD // NS
LW = 128
MSG_W = LW

BN = 1000
BE = 2048
NB_N = N_NODES // BN
NB_E = E_PAD // BE

_f32 = jnp.float32
_bf16 = jnp.bfloat16
_i32 = jnp.int32


def _eq_mat(rows, cols, offset=0, dtype=_f32):
    r = lax.broadcasted_iota(_i32, (rows, cols), 0)
    c = lax.broadcasted_iota(_i32, (rows, cols), 1)
    return (c == r + offset).astype(dtype)


def _node_feat_body(x_ref, inp_ref, outp_ref, in_emb_ref, out_emb_ref, h_ref):
    bn = x_ref.shape[0]
    inp = inp_ref[0, 0, :]
    outp = outp_ref[0, 0, :]
    oh_in = (inp[:, None] == lax.broadcasted_iota(_i32, (bn, 100), 1)).astype(_f32)
    oh_out = (outp[:, None] == lax.broadcasted_iota(_i32, (bn, 100), 1)).astype(_f32)
    a = oh_in @ in_emb_ref[...]
    b = oh_out @ out_emb_ref[...]
    h_ref[...] = (x_ref[...] @ _eq_mat(16, LW)
                  + a @ _eq_mat(8, LW, 16)
                  + b @ _eq_mat(8, LW, 24))


def _node_feat(x, inp3, outp3, in_emb, out_emb):
    return pl.pallas_call(
        _node_feat_body,
        grid=(NB_N,),
        in_specs=[
            pl.BlockSpec((BN, IN_FEATS), lambda i: (i, 0)),
            pl.BlockSpec((1, 1, BN), lambda i: (i, 0, 0)),
            pl.BlockSpec((1, 1, BN), lambda i: (i, 0, 0)),
            pl.BlockSpec((100, 8), lambda i: (0, 0)),
            pl.BlockSpec((100, 8), lambda i: (0, 0)),
        ],
        out_specs=pl.BlockSpec((BN, LW), lambda i: (i, 0)),
        out_shape=jax.ShapeDtypeStruct((N_NODES, LW), _f32),
    )(x, inp3, outp3, in_emb, out_emb)


def _make_sc_gather(kch):
    epw = kch * CH

    def body(h_hbm, src_hbm, xj_hbm, idx_v, h_sh, r0, r1, g0, g1, w0, w1):
        cid = lax.axis_index("c")
        sid = lax.axis_index("s")
        wid = sid * NC + cid
        base = wid * epw
        slab = 640

        @pl.when(sid < NS - 1)
        def _():
            pltpu.sync_copy(h_hbm.at[pl.ds(sid * slab, slab)],
                            h_sh.at[pl.ds(sid * slab, slab)])

        @pl.when(sid == NS - 1)
        def _():
            pltpu.sync_copy(
                h_hbm.at[pl.ds((NS - 1) * slab, N_NODES - (NS - 1) * slab)],
                h_sh.at[pl.ds((NS - 1) * slab, N_NODES - (NS - 1) * slab)])

        pltpu.sync_copy(src_hbm.at[wid], idx_v)
        plsc.subcore_barrier()
        bufs = (r0, r1)
        gsems = (g0, g1)
        wsems = (w0, w1)
        nbuf = 2

        def gather(j, b):
            return pltpu.make_async_copy(h_sh.at[idx_v.at[j]], bufs[b], gsems[b])

        def wb(j, b):
            return pltpu.make_async_copy(bufs[b],
                                         xj_hbm.at[pl.ds(base + j * CH, CH)],
                                         wsems[b])

        for b in range(nbuf):
            gather(b, b).start()

        def loop(t, carry):
            j0 = nbuf * t
            for b in range(nbuf):
                gather(j0 + b, b).wait()
                wb(j0 + b, b).start()
            for b in range(nbuf):
                jn = j0 + b + nbuf

                @pl.when(jn < kch)
                def _(b=b, jn=jn):
                    wb(jn - nbuf, b).wait()
                    gather(jn, b).start()

            return carry

        lax.fori_loop(0, kch // nbuf, loop, 0)
        for b in range(nbuf):
            wb(kch - nbuf + b, b).wait()

    return pl.kernel(
        body,
        out_type=jax.ShapeDtypeStruct((NW * epw, LW), _f32),
        mesh=plsc.VectorSubcoreMesh(core_axis_name="c", subcore_axis_name="s"),
        scratch_types=(
            [pltpu.VMEM((kch, CH), _i32),
             pltpu.VMEM_SHARED((N_NODES, LW), _f32)]
            + [pltpu.VMEM((CH, LW), _f32)] * 2
            + [pltpu.SemaphoreType.DMA] * 4
        ),
    )


_make_sc_gather = functools.cache(_make_sc_gather)


def _edge_msg_body(xj_ref, nt_ref, np_ref, sc_ref, nt_emb_ref, np_emb_ref,
                   w1a_ref, w1b_ref, w1c_ref, b1_ref, w2_ref, b2_ref,
                   expand_ref, fold_ref, p1_ref, p2_ref,
                   out_ref, *, n_edges, offset):
    be = xj_ref.shape[0]
    pid = pl.program_id(0)
    nt = nt_ref[...]
    npp = np_ref[...]
    oh_nt = (nt == lax.broadcasted_iota(_i32, (be, 10), 1)).astype(_bf16)
    oh_np = (npp == lax.broadcasted_iota(_i32, (be, 100), 1)).astype(_bf16)
    dotf = functools.partial(jnp.dot, preferred_element_type=_f32)
    a_nt = (nt_emb_ref[...] @ w1a_ref[...]).astype(_bf16)
    a_np = (np_emb_ref[...] @ w1b_ref[...]).astype(_bf16)
    mlp_in = (dotf(oh_nt, a_nt) + dotf(oh_np, a_np)
              + sc_ref[...] * w1c_ref[...] + b1_ref[...])
    mlp_h = jnp.maximum(mlp_in, 0.0).astype(_bf16)
    r = dotf(mlp_h, w2_ref[...]) + b2_ref[...]
    xj = xj_ref[...].astype(_bf16)
    x_exp = dotf(xj, expand_ref[...])
    s = (x_exp * r).astype(_bf16)
    msg = dotf(s, fold_ref[...])
    row0 = offset + pid * be + lax.broadcasted_iota(_i32, (be, 1), 0)
    valid = (row0 < n_edges).astype(_f32)
    out_ref[...] = (msg * valid) @ p1_ref[...] + valid @ p2_ref[...]


def _edge_msg(xj, nt3, np3, sc2, nt_emb, np_emb, w1a, w1b, w1c, b1r, w2, b2r,
              expand_c, fold_c, p1_c, p2_c, offset):
    body = functools.partial(_edge_msg_body, n_edges=N_EDGES, offset=offset)
    ne = xj.shape[0]
    return pl.pallas_call(
        body,
        grid=(ne // BE,),
        in_specs=[
            pl.BlockSpec((BE, LW), lambda i: (i, 0)),
            pl.BlockSpec((BE, 1), lambda i: (i, 0)),
            pl.BlockSpec((BE, 1), lambda i: (i, 0)),
            pl.BlockSpec((BE, 1), lambda i: (i, 0)),
            pl.BlockSpec((10, 8), lambda i: (0, 0)),
            pl.BlockSpec((100, 8), lambda i: (0, 0)),
            pl.BlockSpec((8, 64), lambda i: (0, 0)),
            pl.BlockSpec((8, 64), lambda i: (0, 0)),
            pl.BlockSpec((1, 64), lambda i: (0, 0)),
            pl.BlockSpec((1, 64), lambda i: (0, 0)),
            pl.BlockSpec((64, 512), lambda i: (0, 0)),
            pl.BlockSpec((1, 512), lambda i: (0, 0)),
            pl.BlockSpec((LW, 512), lambda i: (0, 0)),
            pl.BlockSpec((512, HID), lambda i: (0, 0)),
            pl.BlockSpec((HID, MSG_W), lambda i: (0, 0)),
            pl.BlockSpec((1, MSG_W), lambda i: (0, 0)),
        ],
        out_specs=pl.BlockSpec((BE, MSG_W), lambda i: (i, 0)),
        out_shape=jax.ShapeDtypeStruct((ne, MSG_W), _f32),
    )(xj, nt3, np3, sc2, nt_emb, np_emb, w1a, w1b, w1c, b1r, w2, b2r,
      expand_c, fold_c, p1_c, p2_c)


def _make_sc_scatter(kch):
    epw = kch * CH

    def body(msg_hbm, dst_hbm, init_hbm, parts_hbm,
             idx_v, m0_v, m1_v, shared, sem0, sem1):
        cid = lax.axis_index("c")
        sid = lax.axis_index("s")
        wid = sid * NC + cid
        off = cid * N_PAD + sid * ROWS_PER_SUB
        pltpu.sync_copy(init_hbm.at[pl.ds(off, ROWS_PER_SUB)],
                        shared.at[pl.ds(sid * ROWS_PER_SUB, ROWS_PER_SUB)])
        pltpu.sync_copy(dst_hbm.at[wid], idx_v)
        plsc.subcore_barrier()
        base = wid * epw

        def read(j, buf, sem):
            return pltpu.make_async_copy(msg_hbm.at[pl.ds(base + j * CH, CH)],
                                         buf, sem)

        read(0, m0_v, sem0).start()
        read(1, m1_v, sem1).start()

        def loop(t, carry):
            j0 = 2 * t
            j1 = j0 + 1
            read(j0, m0_v, sem0).wait()
            pltpu.sync_copy(m0_v, shared.at[idx_v.at[j0]], add=True)

            @pl.when(j0 + 2 < kch)
            def _():
                read(j0 + 2, m0_v, sem0).start()

            read(j1, m1_v, sem1).wait()
            pltpu.sync_copy(m1_v, shared.at[idx_v.at[j1]], add=True)

            @pl.when(j1 + 2 < kch)
            def _():
                read(j1 + 2, m1_v, sem1).start()

            return carry

        lax.fori_loop(0, kch // 2, loop, 0)
        plsc.subcore_barrier()
        pltpu.sync_copy(shared.at[pl.ds(sid * ROWS_PER_SUB, ROWS_PER_SUB)],
                        parts_hbm.at[pl.ds(off, ROWS_PER_SUB)])

    return pl.kernel(
        body,
        out_type=jax.ShapeDtypeStruct((NC * N_PAD, MSG_W), _f32),
        mesh=plsc.VectorSubcoreMesh(core_axis_name="c", subcore_axis_name="s"),
        scratch_types=[
            pltpu.VMEM((kch, CH), _i32),
            pltpu.VMEM((CH, MSG_W), _f32),
            pltpu.VMEM((CH, MSG_W), _f32),
            pltpu.VMEM_SHARED((N_PAD, MSG_W), _f32),
            pltpu.SemaphoreType.DMA,
            pltpu.SemaphoreType.DMA,
        ],
    )


_make_sc_scatter = functools.cache(_make_sc_scatter)


def _final_body(h_ref, p0_ref, p1_ref, wroot_ref, bconv_ref,
                ws_ref, bs_ref, wnt_ref, bnt_ref, wt_ref, bt_ref,
                wp_ref, bp_ref, o1_ref, o2_ref, o3_ref, o4_ref):
    s = p0_ref[...] + p1_ref[...]
    agg_sum = s @ _eq_mat(MSG_W, HID)
    cnt = s @ _eq_mat(MSG_W, 1, -HID)
    agg = agg_sum / jnp.maximum(cnt, 1.0)
    z = jnp.maximum(h_ref[...] @ wroot_ref[...] + agg + bconv_ref[...], 0.0)
    o1_ref[...] = z @ ws_ref[...] + bs_ref[...]
    o2_ref[...] = z @ wnt_ref[...] + bnt_ref[...]
    o3_ref[...] = z @ wt_ref[...] + bt_ref[...]
    o4_ref[...] = z @ wp_ref[...] + bp_ref[...]


def _final(h, p0, p1, w_root, b_conv_r, w_s, b_s_r, w_nt, b_nt_r,
           w_t, b_t_r, w_p, b_p_r):
    full = lambda a, b: pl.BlockSpec((a, b), lambda i: (0, 0))
    return pl.pallas_call(
        _final_body,
        grid=(NB_N,),
        in_specs=[
            pl.BlockSpec((BN, LW), lambda i: (i, 0)),
            pl.BlockSpec((BN, MSG_W), lambda i: (i, 0)),
            pl.BlockSpec((BN, MSG_W), lambda i: (i, 0)),
            full(LW, HID), full(1, HID),
            full(HID, 16), full(1, 16),
            full(HID, 10), full(1, 10),
            full(HID, 32), full(1, 32),
            full(HID, 1000), full(1, 1000),
        ],
        out_specs=[
            pl.BlockSpec((BN, 16), lambda i: (i, 0)),
            pl.BlockSpec((BN, 10), lambda i: (i, 0)),
            pl.BlockSpec((BN, 32), lambda i: (i, 0)),
            pl.BlockSpec((BN, 1000), lambda i: (i, 0)),
        ],
        out_shape=[
            jax.ShapeDtypeStruct((N_NODES, 16), _f32),
            jax.ShapeDtypeStruct((N_NODES, 10), _f32),
            jax.ShapeDtypeStruct((N_NODES, 32), _f32),
            jax.ShapeDtypeStruct((N_NODES, 1000), _f32),
        ],
    )(h, p0, p1, w_root, b_conv_r, w_s, b_s_r, w_nt, b_nt_r, w_t, b_t_r,
      w_p, b_p_r)


def kernel(x, input_np, output_np, edge_index, edge_sc, edge_np, edge_nt,
           in_np_emb, out_np_emb, edge_np_emb, edge_nt_emb,
           W1, b1, W2, b2, W_root, b_conv,
           W_super, b_super, W_nt, b_nt, W_tags, b_tags, W_primary, b_primary):
    inp3 = input_np.astype(_i32).reshape(NB_N, 1, BN)
    outp3 = output_np.astype(_i32).reshape(NB_N, 1, BN)
    pad = E_PAD - N_EDGES
    zpad_i = jnp.zeros((pad,), _i32)
    srcf = jnp.concatenate([edge_index[0].astype(_i32), zpad_i])
    dstf = jnp.concatenate([edge_index[1].astype(_i32), zpad_i])
    ntf = jnp.concatenate([edge_nt.astype(_i32), zpad_i])
    npf = jnp.concatenate([edge_np.astype(_i32), zpad_i])
    scf = jnp.concatenate([edge_sc.reshape(N_EDGES, 1).astype(_f32),
                           jnp.zeros((pad, 1), _f32)])
    w1a, w1b, w1c = W1[0:8], W1[8:16], W1[16:17]
    w_root_p = jnp.concatenate([W_root, jnp.zeros((LW - NODE_FEAT, HID), _f32)])
    zeros_acc = jnp.zeros((NC * N_PAD, MSG_W), _f32)

    eh = E_PAD // 2
    kh = KCH // 2
    nbh = eh // BE
    b1r, b2r = b1.reshape(1, 64), b2.reshape(1, 512)
    w2b = W2.astype(_bf16)
    ar512 = jnp.arange(512)
    expand_c = (ar512[None, :] // 16 == jnp.arange(LW)[:, None]).astype(_bf16)
    fold_c = (ar512[:, None] % 16 == jnp.arange(HID)[None, :]).astype(_bf16)
    p1_c = (jnp.arange(MSG_W)[None, :] == jnp.arange(HID)[:, None]).astype(_f32)
    p2_c = (jnp.arange(MSG_W)[None, :] == HID).astype(_f32).reshape(1, MSG_W)

    h = _node_feat(x, inp3, outp3, in_np_emb, out_np_emb)
    parts = zeros_acc
    msgs = []
    for lo in (0, eh):
        src3 = lax.dynamic_slice_in_dim(srcf, lo, eh).reshape(NW, kh, CH)
        xj = _make_sc_gather(kh)(h, src3)
        nt3 = lax.dynamic_slice_in_dim(ntf, lo, eh).reshape(eh, 1)
        np3 = lax.dynamic_slice_in_dim(npf, lo, eh).reshape(eh, 1)
        sc2 = lax.dynamic_slice_in_dim(scf, lo, eh)
        msgs.append(_edge_msg(xj, nt3, np3, sc2, edge_nt_emb, edge_np_emb,
                              w1a, w1b, w1c, b1r, w2b, b2r,
                              expand_c, fold_c, p1_c, p2_c, lo))
    for lo, msg in zip((0, eh), msgs):
        dst3 = lax.dynamic_slice_in_dim(dstf, lo, eh).reshape(NW, kh, CH)
        parts = _make_sc_scatter(kh)(msg, dst3, parts)
    p0 = parts[0:N_NODES]
    p1 = parts[N_PAD:N_PAD + N_NODES]
    return _final(h, p0, p1, w_root_p, b_conv.reshape(1, HID),
                  W_super, b_super.reshape(1, 16), W_nt, b_nt.reshape(1, 10),
                  W_tags, b_tags.reshape(1, 32), W_primary,
                  b_primary.reshape(1, 1000))


# speedup vs baseline: 1.1705x; 1.1705x over previous
import functools

import jax
import jax.numpy as jnp
from jax import lax
from jax.experimental import pallas as pl
from jax.experimental.pallas import tpu as pltpu
from jax.experimental.pallas import tpu_sc as plsc

N_NODES = 10000
N_EDGES = 160000
IN_FEATS = 16
HID = 16
NODE_FEAT = 32

NC = 2
NS = 16
NW = NC * NS
CH = 128
KCH = 40
E_PAD = NW * KCH * CH
EPW = KCH * CH
N_PAD = 10240
ROWS_PER_SUB = N_PAD // NS
LW = 128
MSG_W = LW

BN = 1000
BE = 2048
NB_N = N_NODES // BN
NB_E = E_PAD // BE

_f32 = jnp.float32
_bf16 = jnp.bfloat16
_i32 = jnp.int32


def _eq_mat(rows, cols, offset=0, dtype=_f32):
    r = lax.broadcasted_iota(_i32, (rows, cols), 0)
    c = lax.broadcasted_iota(_i32, (rows, cols), 1)
    return (c == r + offset).astype(dtype)


def _node_feat_body(x_ref, inp_ref, outp_ref, in_emb_ref, out_emb_ref, h_ref):
    bn = x_ref.shape[0]
    inp = inp_ref[0, 0, :]
    outp = outp_ref[0, 0, :]
    oh_in = (inp[:, None] == lax.broadcasted_iota(_i32, (bn, 100), 1)).astype(_f32)
    oh_out = (outp[:, None] == lax.broadcasted_iota(_i32, (bn, 100), 1)).astype(_f32)
    a = oh_in @ in_emb_ref[...]
    b = oh_out @ out_emb_ref[...]
    h_ref[...] = (x_ref[...] @ _eq_mat(16, LW)
                  + a @ _eq_mat(8, LW, 16)
                  + b @ _eq_mat(8, LW, 24))


def _node_feat(x, inp3, outp3, in_emb, out_emb):
    return pl.pallas_call(
        _node_feat_body,
        grid=(NB_N,),
        in_specs=[
            pl.BlockSpec((BN, IN_FEATS), lambda i: (i, 0)),
            pl.BlockSpec((1, 1, BN), lambda i: (i, 0, 0)),
            pl.BlockSpec((1, 1, BN), lambda i: (i, 0, 0)),
            pl.BlockSpec((100, 8), lambda i: (0, 0)),
            pl.BlockSpec((100, 8), lambda i: (0, 0)),
        ],
        out_specs=pl.BlockSpec((BN, LW), lambda i: (i, 0)),
        out_shape=jax.ShapeDtypeStruct((N_NODES, LW), _f32),
    )(x, inp3, outp3, in_emb, out_emb)


def _make_sc_gather(kch):
    epw = kch * CH

    def body(h_hbm, src_hbm, xj_hbm, idx_v, h_sh, r0, r1, g0, g1, w0, w1):
        cid = lax.axis_index("c")
        sid = lax.axis_index("s")
        wid = sid * NC + cid
        base = wid * epw
        slab = 640

        @pl.when(sid < NS - 1)
        def _():
            pltpu.sync_copy(h_hbm.at[pl.ds(sid * slab, slab)],
                            h_sh.at[pl.ds(sid * slab, slab)])

        @pl.when(sid == NS - 1)
        def _():
            pltpu.sync_copy(
                h_hbm.at[pl.ds((NS - 1) * slab, N_NODES - (NS - 1) * slab)],
                h_sh.at[pl.ds((NS - 1) * slab, N_NODES - (NS - 1) * slab)])

        pltpu.sync_copy(src_hbm.at[wid], idx_v)
        plsc.subcore_barrier()
        bufs = (r0, r1)
        gsems = (g0, g1)
        wsems = (w0, w1)
        nbuf = 2

        def gather(j, b):
            return pltpu.make_async_copy(h_sh.at[idx_v.at[j]], bufs[b], gsems[b])

        def wb(j, b):
            return pltpu.make_async_copy(bufs[b],
                                         xj_hbm.at[pl.ds(base + j * CH, CH)],
                                         wsems[b])

        for b in range(nbuf):
            gather(b, b).start()

        def loop(t, carry):
            j0 = nbuf * t
            for b in range(nbuf):
                gather(j0 + b, b).wait()
                wb(j0 + b, b).start()
            for b in range(nbuf):
                jn = j0 + b + nbuf

                @pl.when(jn < kch)
                def _(b=b, jn=jn):
                    wb(jn - nbuf, b).wait()
                    gather(jn, b).start()

            return carry

        lax.fori_loop(0, kch // nbuf, loop, 0)
        for b in range(nbuf):
            wb(kch - nbuf + b, b).wait()

    return pl.kernel(
        body,
        out_type=jax.ShapeDtypeStruct((NW * epw, LW), _f32),
        mesh=plsc.VectorSubcoreMesh(core_axis_name="c", subcore_axis_name="s"),
        scratch_types=(
            [pltpu.VMEM((kch, CH), _i32),
             pltpu.VMEM_SHARED((N_NODES, LW), _f32)]
            + [pltpu.VMEM((CH, LW), _f32)] * 2
            + [pltpu.SemaphoreType.DMA] * 4
        ),
    )


_make_sc_gather = functools.cache(_make_sc_gather)


def _edge_msg_body(xj_ref, nt_ref, np_ref, sc_ref, nt_emb_ref, np_emb_ref,
                   w1a_ref, w1b_ref, w1c_ref, b1_ref, w2_ref, b2_ref,
                   out_ref, *, n_edges, offset):
    be = xj_ref.shape[0]
    pid = pl.program_id(0)
    nt = nt_ref[0, 0, :]
    npp = np_ref[0, 0, :]
    oh_nt = (nt[:, None] == lax.broadcasted_iota(_i32, (be, 10), 1)).astype(_f32)
    oh_np = (npp[:, None] == lax.broadcasted_iota(_i32, (be, 100), 1)).astype(_f32)
    a_nt = nt_emb_ref[...] @ w1a_ref[...]
    a_np = np_emb_ref[...] @ w1b_ref[...]
    mlp_in = oh_nt @ a_nt + oh_np @ a_np + sc_ref[...] * w1c_ref[...] + b1_ref[...]
    mlp_h = jnp.maximum(mlp_in, 0.0)
    r = mlp_h @ w2_ref[...] + b2_ref[...]
    xj = xj_ref[...]
    rk = lax.broadcasted_iota(_i32, (LW, 512), 0)
    ck = lax.broadcasted_iota(_i32, (LW, 512), 1)
    expand = (lax.shift_right_logical(ck, 2 + 2) == rk).astype(_f32)
    x_exp = xj @ expand
    s = x_exp * r
    rm = lax.broadcasted_iota(_i32, (512, HID), 0)
    cm = lax.broadcasted_iota(_i32, (512, HID), 1)
    fold = ((rm & 15) == cm).astype(_f32)
    msg = s @ fold
    row0 = offset + pid * be + lax.broadcasted_iota(_i32, (be, 1), 0)
    valid = (row0 < n_edges).astype(_f32)
    out_ref[...] = (msg * valid) @ _eq_mat(HID, MSG_W) + valid @ _eq_mat(1, MSG_W, HID)


def _edge_msg(xj, nt3, np3, sc2, nt_emb, np_emb, w1a, w1b, w1c, b1r, w2, b2r,
              offset):
    body = functools.partial(_edge_msg_body, n_edges=N_EDGES, offset=offset)
    ne = xj.shape[0]
    return pl.pallas_call(
        body,
        grid=(ne // BE,),
        in_specs=[
            pl.BlockSpec((BE, LW), lambda i: (i, 0)),
            pl.BlockSpec((1, 1, BE), lambda i: (i, 0, 0)),
            pl.BlockSpec((1, 1, BE), lambda i: (i, 0, 0)),
            pl.BlockSpec((BE, 1), lambda i: (i, 0)),
            pl.BlockSpec((10, 8), lambda i: (0, 0)),
            pl.BlockSpec((100, 8), lambda i: (0, 0)),
            pl.BlockSpec((8, 64), lambda i: (0, 0)),
            pl.BlockSpec((8, 64), lambda i: (0, 0)),
            pl.BlockSpec((1, 64), lambda i: (0, 0)),
            pl.BlockSpec((1, 64), lambda i: (0, 0)),
            pl.BlockSpec((64, 512), lambda i: (0, 0)),
            pl.BlockSpec((1, 512), lambda i: (0, 0)),
        ],
        out_specs=pl.BlockSpec((BE, MSG_W), lambda i: (i, 0)),
        out_shape=jax.ShapeDtypeStruct((ne, MSG_W), _f32),
    )(xj, nt3, np3, sc2, nt_emb, np_emb, w1a, w1b, w1c, b1r, w2, b2r)


def _make_sc_scatter(kch):
    epw = kch * CH

    def body(msg_hbm, dst_hbm, init_hbm, parts_hbm,
             idx_v, m0_v, m1_v, shared, sem0, sem1):
        cid = lax.axis_index("c")
        sid = lax.axis_index("s")
        wid = sid * NC + cid
        off = cid * N_PAD + sid * ROWS_PER_SUB
        pltpu.sync_copy(init_hbm.at[pl.ds(off, ROWS_PER_SUB)],
                        shared.at[pl.ds(sid * ROWS_PER_SUB, ROWS_PER_SUB)])
        pltpu.sync_copy(dst_hbm.at[wid], idx_v)
        plsc.subcore_barrier()
        base = wid * epw

        def read(j, buf, sem):
            return pltpu.make_async_copy(msg_hbm.at[pl.ds(base + j * CH, CH)],
                                         buf, sem)

        read(0, m0_v, sem0).start()
        read(1, m1_v, sem1).start()

        def loop(t, carry):
            j0 = 2 * t
            j1 = j0 + 1
            read(j0, m0_v, sem0).wait()
            pltpu.sync_copy(m0_v, shared.at[idx_v.at[j0]], add=True)

            @pl.when(j0 + 2 < kch)
            def _():
                read(j0 + 2, m0_v, sem0).start()

            read(j1, m1_v, sem1).wait()
            pltpu.sync_copy(m1_v, shared.at[idx_v.at[j1]], add=True)

            @pl.when(j1 + 2 < kch)
            def _():
                read(j1 + 2, m1_v, sem1).start()

            return carry

        lax.fori_loop(0, kch // 2, loop, 0)
        plsc.subcore_barrier()
        pltpu.sync_copy(shared.at[pl.ds(sid * ROWS_PER_SUB, ROWS_PER_SUB)],
                        parts_hbm.at[pl.ds(off, ROWS_PER_SUB)])

    return pl.kernel(
        body,
        out_type=jax.ShapeDtypeStruct((NC * N_PAD, MSG_W), _f32),
        mesh=plsc.VectorSubcoreMesh(core_axis_name="c", subcore_axis_name="s"),
        scratch_types=[
            pltpu.VMEM((kch, CH), _i32),
            pltpu.VMEM((CH, MSG_W), _f32),
            pltpu.VMEM((CH, MSG_W), _f32),
            pltpu.VMEM_SHARED((N_PAD, MSG_W), _f32),
            pltpu.SemaphoreType.DMA,
            pltpu.SemaphoreType.DMA,
        ],
    )


_make_sc_scatter = functools.cache(_make_sc_scatter)


def _final_body(h_ref, p0_ref, p1_ref, wroot_ref, bconv_ref,
                ws_ref, bs_ref, wnt_ref, bnt_ref, wt_ref, bt_ref,
                wp_ref, bp_ref, o1_ref, o2_ref, o3_ref, o4_ref):
    s = p0_ref[...] + p1_ref[...]
    agg_sum = s @ _eq_mat(MSG_W, HID)
    cnt = s @ _eq_mat(MSG_W, 1, -HID)
    agg = agg_sum / jnp.maximum(cnt, 1.0)
    z = jnp.maximum(h_ref[...] @ wroot_ref[...] + agg + bconv_ref[...], 0.0)
    o1_ref[...] = z @ ws_ref[...] + bs_ref[...]
    o2_ref[...] = z @ wnt_ref[...] + bnt_ref[...]
    o3_ref[...] = z @ wt_ref[...] + bt_ref[...]
    o4_ref[...] = z @ wp_ref[...] + bp_ref[...]


def _final(h, p0, p1, w_root, b_conv_r, w_s, b_s_r, w_nt, b_nt_r,
           w_t, b_t_r, w_p, b_p_r):
    full = lambda a, b: pl.BlockSpec((a, b), lambda i: (0, 0))
    return pl.pallas_call(
        _final_body,
        grid=(NB_N,),
        in_specs=[
            pl.BlockSpec((BN, LW), lambda i: (i, 0)),
            pl.BlockSpec((BN, MSG_W), lambda i: (i, 0)),
            pl.BlockSpec((BN, MSG_W), lambda i: (i, 0)),
            full(LW, HID), full(1, HID),
            full(HID, 16), full(1, 16),
            full(HID, 10), full(1, 10),
            full(HID, 32), full(1, 32),
            full(HID, 1000), full(1, 1000),
        ],
        out_specs=[
            pl.BlockSpec((BN, 16), lambda i: (i, 0)),
            pl.BlockSpec((BN, 10), lambda i: (i, 0)),
            pl.BlockSpec((BN, 32), lambda i: (i, 0)),
            pl.BlockSpec((BN, 1000), lambda i: (i, 0)),
        ],
        out_shape=[
            jax.ShapeDtypeStruct((N_NODES, 16), _f32),
            jax.ShapeDtypeStruct((N_NODES, 10), _f32),
            jax.ShapeDtypeStruct((N_NODES, 32), _f32),
            jax.ShapeDtypeStruct((N_NODES, 1000), _f32),
        ],
    )(h, p0, p1, w_root, b_conv_r, w_s, b_s_r, w_nt, b_nt_r, w_t, b_t_r,
      w_p, b_p_r)


def kernel(x, input_np, output_np, edge_index, edge_sc, edge_np, edge_nt,
           in_np_emb, out_np_emb, edge_np_emb, edge_nt_emb,
           W1, b1, W2, b2, W_root, b_conv,
           W_super, b_super, W_nt, b_nt, W_tags, b_tags, W_primary, b_primary):
    inp3 = input_np.astype(_i32).reshape(NB_N, 1, BN)
    outp3 = output_np.astype(_i32).reshape(NB_N, 1, BN)
    pad = E_PAD - N_EDGES
    zpad_i = jnp.zeros((pad,), _i32)
    srcf = jnp.concatenate([edge_index[0].astype(_i32), zpad_i])
    dstf = jnp.concatenate([edge_index[1].astype(_i32), zpad_i])
    ntf = jnp.concatenate([edge_nt.astype(_i32), zpad_i])
    npf = jnp.concatenate([edge_np.astype(_i32), zpad_i])
    scf = jnp.concatenate([edge_sc.reshape(N_EDGES, 1).astype(_f32),
                           jnp.zeros((pad, 1), _f32)])
    w1a, w1b, w1c = W1[0:8], W1[8:16], W1[16:17]
    w_root_p = jnp.concatenate([W_root, jnp.zeros((LW - NODE_FEAT, HID), _f32)])
    zeros_acc = jnp.zeros((NC * N_PAD, MSG_W), _f32)

    eh = E_PAD // 2
    kh = KCH // 2
    nbh = eh // BE
    b1r, b2r = b1.reshape(1, 64), b2.reshape(1, 512)

    h = _node_feat(x, inp3, outp3, in_np_emb, out_np_emb)
    parts = zeros_acc
    msgs = []
    for lo in (0, eh):
        src3 = lax.dynamic_slice_in_dim(srcf, lo, eh).reshape(NW, kh, CH)
        xj = _make_sc_gather(kh)(h, src3)
        nt3 = lax.dynamic_slice_in_dim(ntf, lo, eh).reshape(nbh, 1, BE)
        np3 = lax.dynamic_slice_in_dim(npf, lo, eh).reshape(nbh, 1, BE)
        sc2 = lax.dynamic_slice_in_dim(scf, lo, eh)
        msgs.append(_edge_msg(xj, nt3, np3, sc2, edge_nt_emb, edge_np_emb,
                              w1a, w1b, w1c, b1r, W2, b2r, lo))
    for lo, msg in zip((0, eh), msgs):
        dst3 = lax.dynamic_slice_in_dim(dstf, lo, eh).reshape(NW, kh, CH)
        parts = _make_sc_scatter(kh)(msg, dst3, parts)
    p0 = parts[0:N_NODES]
    p1 = parts[N_PAD:N_PAD + N_NODES]
    return _final(h, p0, p1, w_root_p, b_conv.reshape(1, HID),
                  W_super, b_super.reshape(1, 16), W_nt, b_nt.reshape(1, 10),
                  W_tags, b_tags.reshape(1, 32), W_primary,
                  b_primary.reshape(1, 1000))
